# baseline (speedup 1.0000x reference)
import dataclasses
import functools

import jax
import jax.numpy as jnp
from jax import lax
from jax.experimental import pallas as pl
from jax.experimental.pallas import tpu as pltpu
from jax.experimental.pallas import tpu_sc as plsc

N = 10000
E = 160000
FEAT = 256
D_MODEL = 256
DEGREE = 256

NC = 2
NS = 16
NW = NC * NS
EPW = E // NW
LANES = 16
NVEC = (EPW + LANES - 1) // LANES
TAIL = EPW - (NVEC - 1) * LANES
NB = 10
BN = N // NB


def _sc_bincount(col):
    mesh = plsc.VectorSubcoreMesh(core_axis_name="c", subcore_axis_name="s")
    cp = pltpu.CompilerParams(use_tc_tiling_on_sc=False)
    if "needs_layout_passes" in pltpu.CompilerParams.__dataclass_fields__:
        cp = dataclasses.replace(cp, needs_layout_passes=False)

    @functools.partial(
        pl.kernel,
        mesh=mesh,
        compiler_params=cp,
        out_type=jax.ShapeDtypeStruct((NB, NW, BN), jnp.int32),
        scratch_types=[
            pltpu.VMEM((NVEC * LANES,), jnp.int32),
            pltpu.VMEM((N,), jnp.int32),
            pltpu.SemaphoreType.DMA,
        ],
    )
    def bincount_kernel(edge_hbm, out_hbm, idx_v, hist_v, sem):
        wid = lax.axis_index("s") * NC + lax.axis_index("c")
        base = wid * EPW
        zeros16 = jnp.zeros((LANES,), jnp.int32)
        ones16 = jnp.ones((LANES,), jnp.int32)
        lane = lax.iota(jnp.int32, LANES)

        in_cp = pltpu.async_copy(
            edge_hbm.at[1, pl.ds(base, EPW)], idx_v.at[pl.ds(0, EPW)], sem)

        @pl.loop(0, N // LANES)
        def _(i):
            hist_v[pl.ds(i * LANES, LANES)] = zeros16

        in_cp.wait()

        @pl.loop(0, NVEC - 1)
        def _(i):
            v = idx_v[pl.ds(i * LANES, LANES)]
            plsc.addupdate_scatter(hist_v, [v], ones16)

        tail0 = (NVEC - 1) * LANES
        t = idx_v[pl.ds(tail0, LANES)]
        t = jnp.where(lane < TAIL, t, 0)
        plsc.addupdate_scatter(hist_v, [t], ones16, mask=lane < TAIL)

        copies = [
            pltpu.async_copy(hist_v.at[pl.ds(i * BN, BN)], out_hbm.at[i, wid],
                             sem)
            for i in range(NB)
        ]
        for c in copies:
            c.wait()

    return bincount_kernel(col)


def _tc_body(x_ref, hist_ref, w_ref, b_ref, t_ref, o_ref):
    deg = jnp.sum(hist_ref[0], axis=0)
    deg = jnp.minimum(deg, DEGREE - 1)
    iota_d = lax.broadcasted_iota(jnp.int32, (DEGREE, BN), 0)
    onehot_t = (iota_d == deg[None, :]).astype(jnp.float32)
    add = lax.dot_general(onehot_t, t_ref[...], (((0,), (0,)), ((), ())),
                          preferred_element_type=jnp.float32)
    node = lax.dot_general(x_ref[...], w_ref[...], (((1,), (1,)), ((), ())),
                           preferred_element_type=jnp.float32)
    o_ref[...] = node + add + b_ref[...]


def _tc_combine(x, hist3, W, b2, deg_table):
    return pl.pallas_call(
        _tc_body,
        grid=(NB,),
        in_specs=[
            pl.BlockSpec((BN, FEAT), lambda i: (i, 0)),
            pl.BlockSpec((1, NW, BN), lambda i: (i, 0, 0)),
            pl.BlockSpec((D_MODEL, FEAT), lambda i: (0, 0)),
            pl.BlockSpec((1, D_MODEL), lambda i: (0, 0)),
            pl.BlockSpec((DEGREE, D_MODEL), lambda i: (0, 0)),
        ],
        out_specs=pl.BlockSpec((BN, D_MODEL), lambda i: (i, 0)),
        out_shape=jax.ShapeDtypeStruct((N, D_MODEL), jnp.float32),
    )(x, hist3, W, b2, deg_table)


def _tc_matmul_body(x_ref, w_ref, b_ref, o_ref):
    node = lax.dot_general(x_ref[...], w_ref[...], (((1,), (1,)), ((), ())),
                           preferred_element_type=jnp.float32)
    o_ref[...] = node + b_ref[...]


def _tc_add_body(n_ref, hist_ref, t_ref, o_ref):
    deg = jnp.sum(hist_ref[0], axis=0)
    deg = jnp.minimum(deg, DEGREE - 1)
    iota_d = lax.broadcasted_iota(jnp.int32, (DEGREE, BN), 0)
    onehot_t = (iota_d == deg[None, :]).astype(jnp.float32)
    add = lax.dot_general(onehot_t, t_ref[...], (((0,), (0,)), ((), ())),
                          preferred_element_type=jnp.float32)
    o_ref[...] = n_ref[...] + add


def kernel(x, edge_index, W, b, deg_table):
    hist3 = _sc_bincount(edge_index)
    node = pl.pallas_call(
        _tc_matmul_body,
        grid=(NB,),
        in_specs=[
            pl.BlockSpec((BN, FEAT), lambda i: (i, 0)),
            pl.BlockSpec((D_MODEL, FEAT), lambda i: (0, 0)),
            pl.BlockSpec((1, D_MODEL), lambda i: (0, 0)),
        ],
        out_specs=pl.BlockSpec((BN, D_MODEL), lambda i: (i, 0)),
        out_shape=jax.ShapeDtypeStruct((N, D_MODEL), jnp.float32),
    )(x, W, b.reshape(1, D_MODEL))
    return pl.pallas_call(
        _tc_add_body,
        grid=(NB,),
        in_specs=[
            pl.BlockSpec((BN, D_MODEL), lambda i: (i, 0)),
            pl.BlockSpec((1, NW, BN), lambda i: (i, 0, 0)),
            pl.BlockSpec((DEGREE, D_MODEL), lambda i: (0, 0)),
        ],
        out_specs=pl.BlockSpec((BN, D_MODEL), lambda i: (i, 0)),
        out_shape=jax.ShapeDtypeStruct((N, D_MODEL), jnp.float32),
    )(node, hist3, deg_table)


# speedup vs baseline: 1.1286x; 1.1286x over previous
import dataclasses
import functools

import jax
import jax.numpy as jnp
from jax import lax
from jax.experimental import pallas as pl
from jax.experimental.pallas import tpu as pltpu
from jax.experimental.pallas import tpu_sc as plsc

N = 10000
E = 160000
FEAT = 256
D_MODEL = 256
DEGREE = 256

NC = 2
NS = 16
NW = NC * NS
EPW = E // NW
LANES = 16
NVEC = (EPW + LANES - 1) // LANES
TAIL = EPW - (NVEC - 1) * LANES
NB = 10
BN = N // NB


def _sc_bincount(col):
    mesh = plsc.VectorSubcoreMesh(core_axis_name="c", subcore_axis_name="s")
    cp = pltpu.CompilerParams(use_tc_tiling_on_sc=False)
    if "needs_layout_passes" in pltpu.CompilerParams.__dataclass_fields__:
        cp = dataclasses.replace(cp, needs_layout_passes=False)

    @functools.partial(
        pl.kernel,
        mesh=mesh,
        compiler_params=cp,
        out_type=jax.ShapeDtypeStruct((NB, NW, BN), jnp.int32),
        scratch_types=[
            pltpu.VMEM((NVEC * LANES,), jnp.int32),
            pltpu.VMEM((N,), jnp.int32),
            pltpu.SemaphoreType.DMA,
        ],
    )
    def bincount_kernel(edge_hbm, out_hbm, idx_v, hist_v, sem):
        wid = lax.axis_index("s") * NC + lax.axis_index("c")
        base = wid * EPW
        zeros16 = jnp.zeros((LANES,), jnp.int32)
        ones16 = jnp.ones((LANES,), jnp.int32)
        lane = lax.iota(jnp.int32, LANES)

        in_cp = pltpu.async_copy(
            edge_hbm.at[1, pl.ds(base, EPW)], idx_v.at[pl.ds(0, EPW)], sem)

        @pl.loop(0, N // LANES)
        def _(i):
            hist_v[pl.ds(i * LANES, LANES)] = zeros16

        in_cp.wait()

        @pl.loop(0, NVEC - 1)
        def _(i):
            v = idx_v[pl.ds(i * LANES, LANES)]
            plsc.addupdate_scatter(hist_v, [v], ones16)

        tail0 = (NVEC - 1) * LANES
        t = idx_v[pl.ds(tail0, LANES)]
        t = jnp.where(lane < TAIL, t, 0)
        plsc.addupdate_scatter(hist_v, [t], ones16, mask=lane < TAIL)

        copies = [
            pltpu.async_copy(hist_v.at[pl.ds(i * BN, BN)], out_hbm.at[i, wid],
                             sem)
            for i in range(NB)
        ]
        for c in copies:
            c.wait()

    return bincount_kernel(col)


def _tc_body(x_ref, hist_ref, w_ref, b_ref, t_ref, o_ref):
    deg = jnp.sum(hist_ref[0], axis=0)
    deg = jnp.minimum(deg, DEGREE - 1)
    iota_d = lax.broadcasted_iota(jnp.int32, (DEGREE, BN), 0)
    onehot_t = (iota_d == deg[None, :]).astype(jnp.float32)
    add = lax.dot_general(onehot_t, t_ref[...], (((0,), (0,)), ((), ())),
                          preferred_element_type=jnp.float32)
    node = lax.dot_general(x_ref[...], w_ref[...], (((1,), (1,)), ((), ())),
                           preferred_element_type=jnp.float32)
    o_ref[...] = node + add + b_ref[...]


def _tc_combine(x, hist3, W, b2, deg_table):
    return pl.pallas_call(
        _tc_body,
        grid=(NB,),
        in_specs=[
            pl.BlockSpec((BN, FEAT), lambda i: (i, 0)),
            pl.BlockSpec((1, NW, BN), lambda i: (i, 0, 0)),
            pl.BlockSpec((D_MODEL, FEAT), lambda i: (0, 0)),
            pl.BlockSpec((1, D_MODEL), lambda i: (0, 0)),
            pl.BlockSpec((DEGREE, D_MODEL), lambda i: (0, 0)),
        ],
        out_specs=pl.BlockSpec((BN, D_MODEL), lambda i: (i, 0)),
        out_shape=jax.ShapeDtypeStruct((N, D_MODEL), jnp.float32),
    )(x, hist3, W, b2, deg_table)


def _tc_matmul_body(x_ref, w_ref, b_ref, o_ref):
    node = lax.dot_general(x_ref[...], w_ref[...], (((1,), (1,)), ((), ())),
                           preferred_element_type=jnp.float32)
    o_ref[...] = node + b_ref[...]


def _tc_add_body(n_ref, hist_ref, t_ref, o_ref):
    deg = jnp.sum(hist_ref[0], axis=0)
    deg = jnp.minimum(deg, DEGREE - 1)
    iota_d = lax.broadcasted_iota(jnp.int32, (DEGREE, BN), 0)
    onehot_t = (iota_d == deg[None, :]).astype(jnp.float32)
    add = lax.dot_general(onehot_t, t_ref[...], (((0,), (0,)), ((), ())),
                          preferred_element_type=jnp.float32)
    o_ref[...] = n_ref[...] + add


def kernel(x, edge_index, W, b, deg_table):
    hist3 = _sc_bincount(edge_index)
    return _tc_combine(x, hist3, W, b.reshape(1, D_MODEL), deg_table)


# speedup vs baseline: 1.1516x; 1.0204x over previous
import dataclasses
import functools

import jax
import jax.numpy as jnp
from jax import lax
from jax.experimental import pallas as pl
from jax.experimental.pallas import tpu as pltpu
from jax.experimental.pallas import tpu_sc as plsc

N = 10000
E = 160000
FEAT = 256
D_MODEL = 256
DEGREE = 256

NC = 2
NS = 16
NW = NC * NS
EPW = E // NW
LANES = 16
NVEC = (EPW + LANES - 1) // LANES
TAIL = EPW - (NVEC - 1) * LANES
NB = 10
BN = N // NB


def _sc_bincount(edge_index):
    mesh = plsc.VectorSubcoreMesh(core_axis_name="c", subcore_axis_name="s")
    cp = pltpu.CompilerParams(use_tc_tiling_on_sc=False)
    if "needs_layout_passes" in pltpu.CompilerParams.__dataclass_fields__:
        cp = dataclasses.replace(cp, needs_layout_passes=False)

    @functools.partial(
        pl.kernel,
        mesh=mesh,
        compiler_params=cp,
        out_type=jax.ShapeDtypeStruct((NW, N), jnp.int32),
        scratch_types=[
            pltpu.VMEM((NVEC * LANES,), jnp.int32),
            pltpu.VMEM((N,), jnp.int32),
            pltpu.SemaphoreType.DMA,
        ],
    )
    def bincount_kernel(edge_hbm, out_hbm, idx_v, hist_v, sem):
        wid = lax.axis_index("s") * NC + lax.axis_index("c")
        base = wid * EPW
        zeros16 = jnp.zeros((LANES,), jnp.int32)
        ones16 = jnp.ones((LANES,), jnp.int32)
        lane = lax.iota(jnp.int32, LANES)

        in_cp = pltpu.async_copy(
            edge_hbm.at[1, pl.ds(base, EPW)], idx_v.at[pl.ds(0, EPW)], sem)

        @pl.loop(0, N // LANES)
        def _(i):
            hist_v[pl.ds(i * LANES, LANES)] = zeros16

        in_cp.wait()

        @pl.loop(0, NVEC - 1)
        def _(i):
            v = idx_v[pl.ds(i * LANES, LANES)]
            plsc.addupdate_scatter(hist_v, [v], ones16)

        tail0 = (NVEC - 1) * LANES
        t = idx_v[pl.ds(tail0, LANES)]
        t = jnp.where(lane < TAIL, t, 0)
        plsc.addupdate_scatter(hist_v, [t], ones16, mask=lane < TAIL)

        pltpu.sync_copy(hist_v, out_hbm.at[wid])

    return bincount_kernel(edge_index)


def _tc_body(x_ref, deg_ref, w_ref, b_ref, t_ref, o_ref):
    deg = deg_ref[0, 0]
    iota_d = lax.broadcasted_iota(jnp.int32, (DEGREE, BN), 0)
    onehot_t = (iota_d == deg[None, :]).astype(jnp.float32)
    add = lax.dot_general(onehot_t, t_ref[...], (((0,), (0,)), ((), ())),
                          preferred_element_type=jnp.float32)
    node = lax.dot_general(x_ref[...], w_ref[...], (((1,), (1,)), ((), ())),
                           preferred_element_type=jnp.float32)
    o_ref[...] = node + add + b_ref[...]


def _tc_combine(x, deg3, W, b2, deg_table):
    return pl.pallas_call(
        _tc_body,
        grid=(NB,),
        in_specs=[
            pl.BlockSpec((BN, FEAT), lambda i: (i, 0)),
            pl.BlockSpec((1, 1, BN), lambda i: (i, 0, 0)),
            pl.BlockSpec((D_MODEL, FEAT), lambda i: (0, 0)),
            pl.BlockSpec((1, D_MODEL), lambda i: (0, 0)),
            pl.BlockSpec((DEGREE, D_MODEL), lambda i: (0, 0)),
        ],
        out_specs=pl.BlockSpec((BN, D_MODEL), lambda i: (i, 0)),
        out_shape=jax.ShapeDtypeStruct((N, D_MODEL), jnp.float32),
    )(x, deg3, W, b2, deg_table)


def kernel(x, edge_index, W, b, deg_table):
    hist = _sc_bincount(edge_index)
    deg3 = jnp.minimum(hist.sum(axis=0), DEGREE - 1).reshape(NB, 1, BN)
    return _tc_combine(x, deg3, W, b.reshape(1, D_MODEL), deg_table)


# speedup vs baseline: 1.2355x; 1.0728x over previous
import dataclasses
import functools

import jax
import jax.numpy as jnp
from jax import lax
from jax.experimental import pallas as pl
from jax.experimental.pallas import tpu as pltpu
from jax.experimental.pallas import tpu_sc as plsc

N = 10000
E = 160000
FEAT = 256
D_MODEL = 256
DEGREE = 256

NC = 2
NS = 16
NW = NC * NS
LANES = 16
NTILES = E // 128
TPW = NTILES // NW
REM = NTILES - TPW * NW
HR = 79
NB = 10
BN = N // NB


def _sc_bincount(edge_index):
    mesh = plsc.VectorSubcoreMesh(core_axis_name="c", subcore_axis_name="s")
    cp = pltpu.CompilerParams()
    if "needs_layout_passes" in pltpu.CompilerParams.__dataclass_fields__:
        cp = dataclasses.replace(cp, needs_layout_passes=False)

    @functools.partial(
        pl.kernel,
        mesh=mesh,
        compiler_params=cp,
        out_type=jax.ShapeDtypeStruct((NW, HR, 128), jnp.int32),
        scratch_types=[
            pltpu.VMEM((TPW + 1, 2, 128), jnp.int32),
            pltpu.VMEM((HR, 128), jnp.int32),
            pltpu.SemaphoreType.DMA,
        ],
    )
    def bincount_kernel(edge_hbm, out_hbm, tiles_v, hist_v, sem):
        wid = lax.axis_index("s") * NC + lax.axis_index("c")
        t0 = wid * TPW
        zeros16 = jnp.zeros((LANES,), jnp.int32)
        ones16 = jnp.ones((LANES,), jnp.int32)

        copies = [
            pltpu.async_copy(
                edge_hbm.at[:, pl.ds((t0 + k) * 128, 128)], tiles_v.at[k], sem)
            for k in range(TPW)
        ]
        extra = wid < REM
        extra_cp = pltpu.make_async_copy(
            edge_hbm.at[:, pl.ds((NW * TPW + jnp.minimum(wid, REM - 1)) * 128,
                                 128)],
            tiles_v.at[TPW], sem)

        @pl.when(extra)
        def _():
            extra_cp.start()

        @pl.loop(0, HR)
        def _(r):
            @pl.loop(0, 128 // LANES)
            def _(c):
                hist_v[r, pl.ds(c * LANES, LANES)] = zeros16

        for c in copies:
            c.wait()

        @pl.loop(0, TPW * 8)
        def _(i):
            v = tiles_v[i // 8, 1, pl.ds((i % 8) * LANES, LANES)]
            plsc.addupdate_scatter(hist_v, [v >> 7, v & 127], ones16)

        @pl.when(extra)
        def _():
            extra_cp.wait()

            @pl.loop(0, 8)
            def _(j):
                v = tiles_v[TPW, 1, pl.ds(j * LANES, LANES)]
                plsc.addupdate_scatter(hist_v, [v >> 7, v & 127], ones16)

        pltpu.sync_copy(hist_v, out_hbm.at[wid])

    return bincount_kernel(edge_index)


def _tc_body(x_ref, deg_ref, w_ref, b_ref, t_ref, o_ref):
    deg = deg_ref[0, 0]
    iota_d = lax.broadcasted_iota(jnp.int32, (DEGREE, BN), 0)
    onehot_t = (iota_d == deg[None, :]).astype(jnp.float32)
    add = lax.dot_general(onehot_t, t_ref[...], (((0,), (0,)), ((), ())),
                          preferred_element_type=jnp.float32)
    node = lax.dot_general(x_ref[...], w_ref[...], (((1,), (1,)), ((), ())),
                           preferred_element_type=jnp.float32)
    o_ref[...] = node + add + b_ref[...]


def _tc_combine(x, deg3, W, b2, deg_table):
    return pl.pallas_call(
        _tc_body,
        grid=(NB,),
        in_specs=[
            pl.BlockSpec((BN, FEAT), lambda i: (i, 0)),
            pl.BlockSpec((1, 1, BN), lambda i: (i, 0, 0)),
            pl.BlockSpec((D_MODEL, FEAT), lambda i: (0, 0)),
            pl.BlockSpec((1, D_MODEL), lambda i: (0, 0)),
            pl.BlockSpec((DEGREE, D_MODEL), lambda i: (0, 0)),
        ],
        out_specs=pl.BlockSpec((BN, D_MODEL), lambda i: (i, 0)),
        out_shape=jax.ShapeDtypeStruct((N, D_MODEL), jnp.float32),
    )(x, deg3, W, b2, deg_table)


def kernel(x, edge_index, W, b, deg_table):
    hist = _sc_bincount(edge_index)
    deg = jnp.minimum(hist.sum(axis=0), DEGREE - 1)
    deg3 = deg.reshape(HR * 128)[:N].reshape(NB, 1, BN)
    return _tc_combine(x, deg3, W, b.reshape(1, D_MODEL), deg_table)
